# baseline (speedup 1.0000x reference)
import functools

import jax
import jax.numpy as jnp
import numpy as np
from jax import lax
from jax.experimental import pallas as pl
from jax.experimental.pallas import tpu as pltpu
from jax.experimental.pallas import tpu_sc as plsc

_ERROR_PROB = 0.01
_B, _L, _V = 2048, 50, 128
_ROWS = _B * _L
_INV = 1.0 / (_V - 2)

_NC, _NS = 2, 16
_NW = _NC * _NS
_BPW = _B // _NW


def _threefry2x32_np(k1, k2, x0, x1):
    def rotl(x, r):
        return ((x << np.uint32(r)) | (x >> np.uint32(32 - r))).astype(np.uint32)

    ks = [np.uint32(k1), np.uint32(k2),
          np.uint32(np.uint32(k1) ^ np.uint32(k2) ^ np.uint32(0x1BD11BDA))]
    x0 = (x0 + ks[0]).astype(np.uint32)
    x1 = (x1 + ks[1]).astype(np.uint32)
    rots = [(13, 15, 26, 6), (17, 29, 16, 24)]
    for i in range(5):
        for r in rots[i % 2]:
            x0 = (x0 + x1).astype(np.uint32)
            x1 = rotl(x1, r)
            x1 = (x1 ^ x0).astype(np.uint32)
        x0 = (x0 + ks[(i + 1) % 3]).astype(np.uint32)
        x1 = (x1 + ks[(i + 2) % 3] + np.uint32(i + 1)).astype(np.uint32)
    return x0, x1


@functools.cache
def _row_mask_f32() -> np.ndarray:
    n = _ROWS * (_V - 1)
    lo = np.arange(n, dtype=np.uint32)
    hi = np.zeros(n, np.uint32)
    a, b = _threefry2x32_np(np.uint32(0), np.uint32(42), hi, lo)
    bits = a ^ b
    fbits = ((bits >> np.uint32(9)) | np.uint32(0x3F800000)).view(np.float32)
    u = fbits - np.float32(1.0)
    mask = np.any(u.reshape(_ROWS, _V - 1) < np.float32(_ERROR_PROB), axis=1)
    return np.ascontiguousarray(mask.astype(np.float32).reshape(_B, _L, 1))


def _tc_body(mask_ref, m_ref, p_ref, mo_ref, po_ref):
    m = m_ref[...]
    p = p_ref[...]
    mask = mask_ref[...]
    m0 = m[:, :, :1]
    p0 = p[:, :, :1]
    repl = (1.0 - m - m0) * _INV
    m_new = jnp.where(mask > 0.5, repl, m)
    p_new = p * (1.0 - _ERROR_PROB) + (1.0 - p - p0) * (_ERROR_PROB * _INV)
    col = jax.lax.broadcasted_iota(jnp.int32, m.shape, 2)
    is0 = col == 0
    mo_ref[...] = jnp.where(is0, m, m_new)
    po_ref[...] = jnp.where(is0, p, p_new)


def _tc_transform(mask, messages, probs):
    b, l, v = messages.shape
    bblk = 128
    grid = b // bblk
    blk = pl.BlockSpec((bblk, l, v), lambda i: (i, 0, 0))
    out = jax.ShapeDtypeStruct((b, l, v), jnp.float32)
    return pl.pallas_call(
        _tc_body,
        grid=(grid,),
        in_specs=[pl.BlockSpec((bblk, l, 1), lambda i: (i, 0, 0)), blk, blk],
        out_specs=[blk, blk],
        out_shape=[out, out],
    )(mask, messages, probs)


def _make_sc_copy_kernel():
    mesh = plsc.VectorSubcoreMesh(core_axis_name="c", subcore_axis_name="s")
    out = jax.ShapeDtypeStruct((_B, _L, _V), jnp.float32)

    @functools.partial(
        pl.kernel,
        out_type=[out, out],
        mesh=mesh,
    )
    def k(m_hbm, p_hbm, mc_hbm, pc_hbm):
        wid = lax.axis_index("s") * _NC + lax.axis_index("c")
        b0 = wid * _BPW
        sl = pl.ds(b0, _BPW)
        pltpu.sync_copy(m_hbm.at[sl], mc_hbm.at[sl])
        pltpu.sync_copy(p_hbm.at[sl], pc_hbm.at[sl])

    return k


def kernel(messages, probs):
    mask = jnp.asarray(_row_mask_f32())
    m1, p1 = _tc_transform(mask, messages, probs)
    mc, pc = _make_sc_copy_kernel()(messages, probs)
    return (m1, p1, mc, pc)


# speedup vs baseline: 10.1169x; 10.1169x over previous
import functools

import jax
import jax.numpy as jnp
import numpy as np
from jax import lax
from jax.experimental import pallas as pl
from jax.experimental.pallas import tpu as pltpu
from jax.experimental.pallas import tpu_sc as plsc

_ERROR_PROB = 0.01
_B, _L, _V = 2048, 50, 128
_ROWS = _B * _L
_INV = 1.0 / (_V - 2)

_NC, _NS = 2, 16
_NW = _NC * _NS
_BPW = _B // _NW


def _threefry2x32_np(k1, k2, x0, x1):
    def rotl(x, r):
        return ((x << np.uint32(r)) | (x >> np.uint32(32 - r))).astype(np.uint32)

    ks = [np.uint32(k1), np.uint32(k2),
          np.uint32(np.uint32(k1) ^ np.uint32(k2) ^ np.uint32(0x1BD11BDA))]
    x0 = (x0 + ks[0]).astype(np.uint32)
    x1 = (x1 + ks[1]).astype(np.uint32)
    rots = [(13, 15, 26, 6), (17, 29, 16, 24)]
    for i in range(5):
        for r in rots[i % 2]:
            x0 = (x0 + x1).astype(np.uint32)
            x1 = rotl(x1, r)
            x1 = (x1 ^ x0).astype(np.uint32)
        x0 = (x0 + ks[(i + 1) % 3]).astype(np.uint32)
        x1 = (x1 + ks[(i + 2) % 3] + np.uint32(i + 1)).astype(np.uint32)
    return x0, x1


@functools.cache
def _row_mask_f32() -> np.ndarray:
    n = _ROWS * (_V - 1)
    lo = np.arange(n, dtype=np.uint32)
    hi = np.zeros(n, np.uint32)
    a, b = _threefry2x32_np(np.uint32(0), np.uint32(42), hi, lo)
    bits = a ^ b
    fbits = ((bits >> np.uint32(9)) | np.uint32(0x3F800000)).view(np.float32)
    u = fbits - np.float32(1.0)
    mask = np.any(u.reshape(_ROWS, _V - 1) < np.float32(_ERROR_PROB), axis=1)
    return np.ascontiguousarray(mask.astype(np.float32).reshape(_B, _L, 1))


def _tc_body(mask_ref, m_ref, p_ref, mo_ref, po_ref):
    m = m_ref[...]
    p = p_ref[...]
    mask = mask_ref[...]
    m0 = m[:, :, :1]
    p0 = p[:, :, :1]
    repl = (1.0 - m - m0) * _INV
    m_new = jnp.where(mask > 0.5, repl, m)
    p_new = p * (1.0 - _ERROR_PROB) + (1.0 - p - p0) * (_ERROR_PROB * _INV)
    col = jax.lax.broadcasted_iota(jnp.int32, m.shape, 2)
    is0 = col == 0
    mo_ref[...] = jnp.where(is0, m, m_new)
    po_ref[...] = jnp.where(is0, p, p_new)


def _tc_transform(mask, messages, probs):
    b, l, v = messages.shape
    bblk = 32
    grid = b // bblk
    blk = pl.BlockSpec((bblk, l, v), lambda i: (i, 0, 0))
    out = jax.ShapeDtypeStruct((b, l, v), jnp.float32)
    return pl.pallas_call(
        _tc_body,
        grid=(grid,),
        in_specs=[pl.BlockSpec((bblk, l, 1), lambda i: (i, 0, 0)), blk, blk],
        out_specs=[blk, blk],
        out_shape=[out, out],
    )(mask, messages, probs)


def _make_sc_copy_kernel():
    mesh = plsc.VectorSubcoreMesh(core_axis_name="c", subcore_axis_name="s")
    out = jax.ShapeDtypeStruct((_B, _L, _V), jnp.float32)

    @functools.partial(
        pl.kernel,
        out_type=[out, out],
        mesh=mesh,
    )
    def k(m_hbm, p_hbm, mc_hbm, pc_hbm):
        wid = lax.axis_index("s") * _NC + lax.axis_index("c")
        b0 = wid * _BPW
        sl = pl.ds(b0, _BPW)
        pltpu.sync_copy(m_hbm.at[sl], mc_hbm.at[sl])
        pltpu.sync_copy(p_hbm.at[sl], pc_hbm.at[sl])

    return k


def kernel(messages, probs):
    mask = jnp.asarray(_row_mask_f32())
    m1, p1 = _tc_transform(mask, messages, probs)
    return (m1, p1, messages, probs)
